# Initial kernel scaffold; baseline (speedup 1.0000x reference)
import jax
import jax.numpy as jnp
from jax.experimental import pallas as pl


def kernel(nodes_C, nodes_D, nodes_in_degree, nodes_out_degree, nodes_loc, nodes_ava, adj, edge_attr, loc_cpu_speed, loc_min_processor_EAT, loc_num_processor, params):
    raise NotImplementedError("write your pallas kernel here")


# speedup vs baseline: 1782.0313x; 1782.0313x over previous
import jax
import jax.numpy as jnp
from jax.experimental import pallas as pl

N = 512
L = 64
HID = 128
_NEG = -1e30


def _layernorm(x, g, b):
    mu = jnp.mean(x, axis=-1, keepdims=True)
    var = jnp.mean((x - mu) ** 2, axis=-1, keepdims=True)
    return (x - mu) * jax.lax.rsqrt(var + 1e-5) * g + b


def _conv_layer(x, maskT, leaT, W, bias, We, wb_o, wb_s, H, C):
    qkvs = jnp.dot(x, W, preferred_element_type=jnp.float32) + bias
    q = qkvs[:, 0 * HID:1 * HID]
    k = qkvs[:, 1 * HID:2 * HID]
    v = qkvs[:, 2 * HID:3 * HID]
    skip = qkvs[:, 3 * HID:4 * HID]
    qWe = q * We
    inv = jnp.float32(1.0 / (C ** 0.5))
    outs = []
    for h in range(H):
        sl = slice(h * C, (h + 1) * C)
        qh = q[:, sl]
        kh = k[:, sl]
        vh = v[:, sl]
        th = jnp.sum(qWe[:, sl], axis=1, keepdims=True)
        s = jax.lax.dot_general(qh, kh, (((1,), (1,)), ((), ())),
                                preferred_element_type=jnp.float32)
        s = (s + leaT * th) * inv
        s = jnp.where(maskT, s, _NEG)
        amax = jnp.max(s, axis=1, keepdims=True)
        amax = jnp.where(amax > -1e29, amax, 0.0)
        ex = jnp.where(maskT, jnp.exp(s - amax), 0.0)
        denom = jnp.sum(ex, axis=1, keepdims=True)
        a = ex / (denom + 1e-16)
        oh = jnp.dot(a, vh, preferred_element_type=jnp.float32)
        w = jnp.sum(a * leaT, axis=1, keepdims=True)
        outs.append(oh + w * We[:, sl])
    out = outs[0] if H == 1 else jnp.concatenate(outs, axis=1)
    blogit = (jnp.sum(out * wb_o, axis=1, keepdims=True)
              + jnp.sum(skip * wb_s, axis=1, keepdims=True))
    beta = jax.nn.sigmoid(blogit)
    return beta * skip + (1.0 - beta) * out


def _body(nfeat, lfeat, adjT, eaT,
          W0, b0, We0, wo0, ws0,
          W1, b1, We1, wo1, ws1,
          W2, b2, We2, wo2, ws2,
          gn, bn, Wl0, bl0, Wl1, bl1, gl, bl,
          Wf0, bf0, Wf1, bf1,
          out_node, out_loc, out_ge, out_lg, out_lat):
    maskT = adjT[:] > 0.0
    leaT = jnp.log1p(eaT[:])
    x = jax.nn.relu(_conv_layer(nfeat[:], maskT, leaT, W0[:], b0[:],
                                We0[:], wo0[:], ws0[:], 4, HID // 4))
    x = jax.nn.relu(_conv_layer(x, maskT, leaT, W1[:], b1[:],
                                We1[:], wo1[:], ws1[:], 4, HID // 4))
    x = _conv_layer(x, maskT, leaT, W2[:], b2[:],
                    We2[:], wo2[:], ws2[:], 1, HID)
    node_embs = _layernorm(x, gn[:], bn[:])
    out_node[:] = node_embs

    h = jax.nn.relu(jnp.dot(lfeat[:], Wl0[:],
                            preferred_element_type=jnp.float32) + bl0[:])
    h = jnp.dot(h, Wl1[:], preferred_element_type=jnp.float32) + bl1[:]
    loc_embs = _layernorm(h, gl[:], bl[:])
    out_loc[:] = loc_embs

    graph_emb = jnp.mean(node_embs, axis=0, keepdims=True)
    loc_global = jnp.mean(loc_embs, axis=0, keepdims=True)
    out_ge[:] = graph_emb
    out_lg[:] = loc_global

    z = jnp.concatenate([graph_emb, loc_global], axis=1)
    z = jax.nn.relu(jnp.dot(z, Wf0[:], preferred_element_type=jnp.float32)
                    + bf0[:])
    out_lat[:] = (jnp.dot(z, Wf1[:], preferred_element_type=jnp.float32)
                  + bf1[:])


def _prep_conv(p, din_pad):
    hc = HID
    W = jnp.concatenate([p['Wq'], p['Wk'], p['Wv'], p['Ws']], axis=1)
    din = W.shape[0]
    if din < din_pad:
        W = jnp.concatenate(
            [W, jnp.zeros((din_pad - din, 4 * hc), jnp.float32)], axis=0)
    bias = jnp.concatenate([p['bq'], p['bk'], p['bv'], p['bs']])[None, :]
    wb = p['Wb'][:, 0]
    wb_o = (wb[:hc] + wb[2 * hc:])[None, :]
    wb_s = (wb[hc:2 * hc] - wb[2 * hc:])[None, :]
    return W, bias, p['We'], wb_o, wb_s


def kernel(nodes_C, nodes_D, nodes_in_degree, nodes_out_degree, nodes_loc,
           nodes_ava, adj, edge_attr, loc_cpu_speed, loc_min_processor_EAT,
           loc_num_processor, params):
    zpad = jnp.zeros((N,), jnp.float32)
    nfeat = jnp.stack([nodes_C, nodes_D, nodes_in_degree, nodes_out_degree,
                       nodes_loc, nodes_ava, zpad, zpad], axis=-1)
    zl = jnp.zeros((L,), jnp.float32)
    lfeat = jnp.stack([loc_cpu_speed, loc_min_processor_EAT,
                       loc_num_processor, zl, zl, zl, zl, zl], axis=-1)

    adjT = adj.T
    eaT = edge_attr.T

    c0 = _prep_conv(params['conv0'], 8)
    c1 = _prep_conv(params['conv1'], HID)
    c2 = _prep_conv(params['conv2'], HID)

    Wl0 = jnp.concatenate(
        [params['loc0']['W'], jnp.zeros((5, HID), jnp.float32)], axis=0)

    args = (nfeat, lfeat, adjT, eaT,
            *c0, *c1, *c2,
            params['ln_node']['g'][None, :], params['ln_node']['b'][None, :],
            Wl0, params['loc0']['b'][None, :],
            params['loc1']['W'], params['loc1']['b'][None, :],
            params['ln_loc']['g'][None, :], params['ln_loc']['b'][None, :],
            params['fus0']['W'], params['fus0']['b'][None, :],
            params['fus1']['W'], params['fus1']['b'][None, :])

    f32 = jnp.float32
    out_shape = [
        jax.ShapeDtypeStruct((N, HID), f32),
        jax.ShapeDtypeStruct((L, HID), f32),
        jax.ShapeDtypeStruct((1, HID), f32),
        jax.ShapeDtypeStruct((1, HID), f32),
        jax.ShapeDtypeStruct((1, HID), f32),
    ]
    node_embs, loc_embs, graph_emb, loc_global, latent = pl.pallas_call(
        _body, out_shape=out_shape)(*args)
    return node_embs, loc_embs, graph_emb, loc_global, latent


# speedup vs baseline: 2651.0710x; 1.4877x over previous
import jax
import jax.numpy as jnp
from jax.experimental import pallas as pl

N = 512
L = 64
HID = 128
_NEG = -1e30


def _layernorm(x, g, b):
    mu = jnp.mean(x, axis=-1, keepdims=True)
    var = jnp.mean((x - mu) ** 2, axis=-1, keepdims=True)
    return (x - mu) * jax.lax.rsqrt(var + 1e-5) * g + b


def _mm(a, b):
    return jnp.dot(a, b, preferred_element_type=jnp.float32)


def _conv_layer(x, maskT, leaT, p, H, C):
    q = _mm(x, p['Wq']) + p['bq']
    k = _mm(x, p['Wk']) + p['bk']
    v = _mm(x, p['Wv']) + p['bv']
    skip = _mm(x, p['Ws']) + p['bs']
    We = p['We']
    qWe = q * We
    inv = 1.0 / (C ** 0.5)
    outs = []
    for h in range(H):
        sl = slice(h * C, (h + 1) * C)
        qh = q[:, sl]
        kh = k[:, sl]
        vh = v[:, sl]
        th = jnp.sum(qWe[:, sl], axis=1, keepdims=True)
        s = jax.lax.dot_general(qh, kh, (((1,), (1,)), ((), ())),
                                preferred_element_type=jnp.float32)
        s = (s + leaT * th) * inv
        s = jnp.where(maskT, s, _NEG)
        amax = jnp.max(s, axis=1, keepdims=True)
        amax = jnp.where(amax > -1e29, amax, 0.0)
        ex = jnp.where(maskT, jnp.exp(s - amax), 0.0)
        denom = jnp.sum(ex, axis=1, keepdims=True)
        a = ex / (denom + 1e-16)
        oh = _mm(a, vh)
        w = jnp.sum(a * leaT, axis=1, keepdims=True)
        outs.append(oh + w * We[:, sl])
    out = outs[0] if H == 1 else jnp.concatenate(outs, axis=1)
    Wb = p['Wb']
    wb_o = Wb[0 * HID:1 * HID, :] + Wb[2 * HID:3 * HID, :]
    wb_s = Wb[1 * HID:2 * HID, :] - Wb[2 * HID:3 * HID, :]
    beta = jax.nn.sigmoid(_mm(out, wb_o) + _mm(skip, wb_s))
    return beta * skip + (1.0 - beta) * out


_CONV_KEYS = ('Wq', 'bq', 'Wk', 'bk', 'Wv', 'bv', 'Ws', 'bs', 'We', 'Wb')


def _body(nfeat, lfeat, adj, ea,
          *refs):
    it = iter(refs)
    convs = []
    for _ in range(3):
        convs.append({kk: next(it)[:] for kk in _CONV_KEYS})
    (gn, bn, Wl0, bl0, Wl1, bl1, gl, bl, Wf0a, Wf0b, bf0, Wf1, bf1,
     out_node, out_loc, out_ge, out_lg, out_lat) = list(it)

    maskT = adj[:].T > 0.0
    leaT = jnp.log1p(ea[:].T)

    x = jax.nn.relu(_conv_layer(nfeat[:], maskT, leaT, convs[0], 4, HID // 4))
    x = jax.nn.relu(_conv_layer(x, maskT, leaT, convs[1], 4, HID // 4))
    x = _conv_layer(x, maskT, leaT, convs[2], 1, HID)
    node_embs = _layernorm(x, gn[:], bn[:])
    out_node[:] = node_embs

    h = jax.nn.relu(_mm(lfeat[:], Wl0[:]) + bl0[:])
    h = _mm(h, Wl1[:]) + bl1[:]
    loc_embs = _layernorm(h, gl[:], bl[:])
    out_loc[:] = loc_embs

    graph_emb = jnp.mean(node_embs, axis=0, keepdims=True)
    loc_global = jnp.mean(loc_embs, axis=0, keepdims=True)
    out_ge[:] = graph_emb
    out_lg[:] = loc_global

    z = jax.nn.relu(_mm(graph_emb, Wf0a[:]) + _mm(loc_global, Wf0b[:])
                    + bf0[:])
    out_lat[:] = _mm(z, Wf1[:]) + bf1[:]


def kernel(nodes_C, nodes_D, nodes_in_degree, nodes_out_degree, nodes_loc,
           nodes_ava, adj, edge_attr, loc_cpu_speed, loc_min_processor_EAT,
           loc_num_processor, params):
    nfeat = jnp.stack([nodes_C, nodes_D, nodes_in_degree, nodes_out_degree,
                       nodes_loc, nodes_ava], axis=-1)
    lfeat = jnp.stack([loc_cpu_speed, loc_min_processor_EAT,
                       loc_num_processor], axis=-1)

    conv_args = []
    for name in ('conv0', 'conv1', 'conv2'):
        p = params[name]
        for kk in _CONV_KEYS:
            a = p[kk]
            conv_args.append(a[None, :] if a.ndim == 1 else a)

    Wf0 = params['fus0']['W']
    args = (nfeat, lfeat, adj, edge_attr, *conv_args,
            params['ln_node']['g'][None, :], params['ln_node']['b'][None, :],
            params['loc0']['W'], params['loc0']['b'][None, :],
            params['loc1']['W'], params['loc1']['b'][None, :],
            params['ln_loc']['g'][None, :], params['ln_loc']['b'][None, :],
            Wf0[:HID], Wf0[HID:], params['fus0']['b'][None, :],
            params['fus1']['W'], params['fus1']['b'][None, :])

    f32 = jnp.float32
    out_shape = [
        jax.ShapeDtypeStruct((N, HID), f32),
        jax.ShapeDtypeStruct((L, HID), f32),
        jax.ShapeDtypeStruct((1, HID), f32),
        jax.ShapeDtypeStruct((1, HID), f32),
        jax.ShapeDtypeStruct((1, HID), f32),
    ]
    node_embs, loc_embs, graph_emb, loc_global, latent = pl.pallas_call(
        _body, out_shape=out_shape)(*args)
    return node_embs, loc_embs, graph_emb, loc_global, latent


# speedup vs baseline: 2812.7652x; 1.0610x over previous
import jax
import jax.numpy as jnp
from jax.experimental import pallas as pl

N = 512
L = 64
HID = 128
_NEG = -1e30


def _layernorm(x, g, b):
    mu = jnp.mean(x, axis=-1, keepdims=True)
    var = jnp.mean((x - mu) ** 2, axis=-1, keepdims=True)
    return (x - mu) * jax.lax.rsqrt(var + 1e-5) * g + b


def _mm(a, b):
    return jnp.dot(a, b, preferred_element_type=jnp.float32)


def _conv_layer(x, maskT, leaT, p, H, C):
    inv = 1.0 / (C ** 0.5)
    q = (_mm(x, p['Wq']) + p['bq']) * inv
    k = _mm(x, p['Wk']) + p['bk']
    v = _mm(x, p['Wv']) + p['bv']
    skip = _mm(x, p['Ws']) + p['bs']
    We = p['We']
    qWe = q * We
    outs = []
    for h in range(H):
        sl = slice(h * C, (h + 1) * C)
        qh = q[:, sl]
        kh = k[:, sl]
        vh = v[:, sl]
        th = jnp.sum(qWe[:, sl], axis=1, keepdims=True)
        s = jax.lax.dot_general(qh, kh, (((1,), (1,)), ((), ())),
                                preferred_element_type=jnp.float32)
        s = jnp.where(maskT, s + leaT * th, _NEG)
        amax = jnp.max(s, axis=1, keepdims=True)
        amax = jnp.where(amax > -1e29, amax, 0.0)
        ex = jnp.exp(s - amax)
        denom = jnp.sum(ex, axis=1, keepdims=True)
        rec = 1.0 / (denom + 1e-16)
        oh = _mm(ex, vh) * rec
        w = jnp.sum(ex * leaT, axis=1, keepdims=True) * rec
        outs.append(oh + w * We[:, sl])
    out = outs[0] if H == 1 else jnp.concatenate(outs, axis=1)
    Wb = p['Wb']
    wb_o = Wb[0 * HID:1 * HID, :] + Wb[2 * HID:3 * HID, :]
    wb_s = Wb[1 * HID:2 * HID, :] - Wb[2 * HID:3 * HID, :]
    beta = jax.nn.sigmoid(_mm(out, wb_o) + _mm(skip, wb_s))
    return beta * skip + (1.0 - beta) * out


_CONV_KEYS = ('Wq', 'bq', 'Wk', 'bk', 'Wv', 'bv', 'Ws', 'bs', 'We', 'Wb')


def _body(nfeat, lfeat, adj, ea,
          *refs):
    it = iter(refs)
    convs = []
    for _ in range(3):
        convs.append({kk: next(it)[:] for kk in _CONV_KEYS})
    (gn, bn, Wl0, bl0, Wl1, bl1, gl, bl, Wf0a, Wf0b, bf0, Wf1, bf1,
     out_node, out_loc, out_ge, out_lg, out_lat) = list(it)

    maskT = adj[:].T > 0.0
    leaT = jnp.log1p(ea[:].T)

    x = jax.nn.relu(_conv_layer(nfeat[:], maskT, leaT, convs[0], 4, HID // 4))
    x = jax.nn.relu(_conv_layer(x, maskT, leaT, convs[1], 4, HID // 4))
    x = _conv_layer(x, maskT, leaT, convs[2], 1, HID)
    node_embs = _layernorm(x, gn[:], bn[:])
    out_node[:] = node_embs

    h = jax.nn.relu(_mm(lfeat[:], Wl0[:]) + bl0[:])
    h = _mm(h, Wl1[:]) + bl1[:]
    loc_embs = _layernorm(h, gl[:], bl[:])
    out_loc[:] = loc_embs

    graph_emb = jnp.mean(node_embs, axis=0, keepdims=True)
    loc_global = jnp.mean(loc_embs, axis=0, keepdims=True)
    out_ge[:] = graph_emb
    out_lg[:] = loc_global

    z = jax.nn.relu(_mm(graph_emb, Wf0a[:]) + _mm(loc_global, Wf0b[:])
                    + bf0[:])
    out_lat[:] = _mm(z, Wf1[:]) + bf1[:]


def kernel(nodes_C, nodes_D, nodes_in_degree, nodes_out_degree, nodes_loc,
           nodes_ava, adj, edge_attr, loc_cpu_speed, loc_min_processor_EAT,
           loc_num_processor, params):
    nfeat = jnp.stack([nodes_C, nodes_D, nodes_in_degree, nodes_out_degree,
                       nodes_loc, nodes_ava], axis=-1)
    lfeat = jnp.stack([loc_cpu_speed, loc_min_processor_EAT,
                       loc_num_processor], axis=-1)

    conv_args = []
    for name in ('conv0', 'conv1', 'conv2'):
        p = params[name]
        for kk in _CONV_KEYS:
            a = p[kk]
            conv_args.append(a[None, :] if a.ndim == 1 else a)

    Wf0 = params['fus0']['W']
    args = (nfeat, lfeat, adj, edge_attr, *conv_args,
            params['ln_node']['g'][None, :], params['ln_node']['b'][None, :],
            params['loc0']['W'], params['loc0']['b'][None, :],
            params['loc1']['W'], params['loc1']['b'][None, :],
            params['ln_loc']['g'][None, :], params['ln_loc']['b'][None, :],
            Wf0[:HID], Wf0[HID:], params['fus0']['b'][None, :],
            params['fus1']['W'], params['fus1']['b'][None, :])

    f32 = jnp.float32
    out_shape = [
        jax.ShapeDtypeStruct((N, HID), f32),
        jax.ShapeDtypeStruct((L, HID), f32),
        jax.ShapeDtypeStruct((1, HID), f32),
        jax.ShapeDtypeStruct((1, HID), f32),
        jax.ShapeDtypeStruct((1, HID), f32),
    ]
    node_embs, loc_embs, graph_emb, loc_global, latent = pl.pallas_call(
        _body, out_shape=out_shape)(*args)
    return node_embs, loc_embs, graph_emb, loc_global, latent


# speedup vs baseline: 3102.8031x; 1.1031x over previous
import jax
import jax.numpy as jnp
from jax.experimental import pallas as pl

N = 512
L = 64
HID = 128
_NEG = -1e30


def _layernorm(x, g, b):
    mu = jnp.mean(x, axis=-1, keepdims=True)
    var = jnp.mean((x - mu) ** 2, axis=-1, keepdims=True)
    return (x - mu) * jax.lax.rsqrt(var + 1e-5) * g + b


def _mm(a, b):
    return jnp.dot(a, b, preferred_element_type=jnp.float32)


def _conv_layer(x, maskT, leaT, p, H, C):
    inv = 1.0 / (C ** 0.5)
    q = (_mm(x, p['Wq']) + p['bq']) * inv
    k = _mm(x, p['Wk']) + p['bk']
    v = _mm(x, p['Wv']) + p['bv']
    skip = _mm(x, p['Ws']) + p['bs']
    We = p['We']
    qWe = q * We
    outs = []
    for h in range(H):
        sl = slice(h * C, (h + 1) * C)
        qh = q[:, sl]
        kh = k[:, sl]
        vh = v[:, sl]
        th = jnp.sum(qWe[:, sl], axis=1, keepdims=True)
        s = jax.lax.dot_general(qh, kh, (((1,), (1,)), ((), ())),
                                preferred_element_type=jnp.float32)
        s = jnp.where(maskT, s + leaT * th, _NEG)
        amax = jnp.max(s, axis=1, keepdims=True)
        amax = jnp.where(amax > -1e29, amax, 0.0)
        ex = jnp.exp(s - amax)
        denom = jnp.sum(ex, axis=1, keepdims=True)
        rec = 1.0 / (denom + 1e-16)
        oh = _mm(ex, vh) * rec
        w = jnp.sum(ex * leaT, axis=1, keepdims=True) * rec
        outs.append(oh + w * We[:, sl])
    out = outs[0] if H == 1 else jnp.concatenate(outs, axis=1)
    Wb = p['Wb']
    wb_o = Wb[0 * HID:1 * HID, :] + Wb[2 * HID:3 * HID, :]
    wb_s = Wb[1 * HID:2 * HID, :] - Wb[2 * HID:3 * HID, :]
    beta = jax.nn.sigmoid(_mm(out, wb_o) + _mm(skip, wb_s))
    return beta * skip + (1.0 - beta) * out


_CONV_KEYS = ('Wq', 'bq', 'Wk', 'bk', 'Wv', 'bv', 'Ws', 'bs', 'We', 'Wb')


def _body(nfeat, lfeat, adj, ea,
          *refs):
    it = iter(refs)
    convs = []
    for _ in range(3):
        convs.append({kk: next(it)[:] for kk in _CONV_KEYS})
    (gn, bn, Wl0, bl0, Wl1, bl1, gl, bl, Wf0, bf0, Wf1, bf1,
     out_node, out_loc, out_ge, out_lg, out_lat) = list(it)

    maskT = adj[:].T > 0.0
    leaT = jnp.log1p(ea[:].T)

    x = jax.nn.relu(_conv_layer(nfeat[:], maskT, leaT, convs[0], 4, HID // 4))
    x = jax.nn.relu(_conv_layer(x, maskT, leaT, convs[1], 4, HID // 4))
    x = _conv_layer(x, maskT, leaT, convs[2], 1, HID)
    node_embs = _layernorm(x, gn[:], bn[:])
    out_node[:] = node_embs

    h = jax.nn.relu(_mm(lfeat[:], Wl0[:]) + bl0[:])
    h = _mm(h, Wl1[:]) + bl1[:]
    loc_embs = _layernorm(h, gl[:], bl[:])
    out_loc[:] = loc_embs

    graph_emb = jnp.mean(node_embs, axis=0, keepdims=True)
    loc_global = jnp.mean(loc_embs, axis=0, keepdims=True)
    out_ge[:] = graph_emb
    out_lg[:] = loc_global

    Wf0m = Wf0[:]
    z = jax.nn.relu(_mm(graph_emb, Wf0m[:HID]) + _mm(loc_global, Wf0m[HID:])
                    + bf0[:])
    out_lat[:] = _mm(z, Wf1[:]) + bf1[:]


def kernel(nodes_C, nodes_D, nodes_in_degree, nodes_out_degree, nodes_loc,
           nodes_ava, adj, edge_attr, loc_cpu_speed, loc_min_processor_EAT,
           loc_num_processor, params):
    nfeat = jnp.stack([nodes_C, nodes_D, nodes_in_degree, nodes_out_degree,
                       nodes_loc, nodes_ava], axis=-1)
    lfeat = jnp.stack([loc_cpu_speed, loc_min_processor_EAT,
                       loc_num_processor], axis=-1)

    conv_args = []
    for name in ('conv0', 'conv1', 'conv2'):
        p = params[name]
        for kk in _CONV_KEYS:
            conv_args.append(p[kk])

    args = (nfeat, lfeat, adj, edge_attr, *conv_args,
            params['ln_node']['g'], params['ln_node']['b'],
            params['loc0']['W'], params['loc0']['b'],
            params['loc1']['W'], params['loc1']['b'],
            params['ln_loc']['g'], params['ln_loc']['b'],
            params['fus0']['W'], params['fus0']['b'],
            params['fus1']['W'], params['fus1']['b'])

    f32 = jnp.float32
    out_shape = [
        jax.ShapeDtypeStruct((N, HID), f32),
        jax.ShapeDtypeStruct((L, HID), f32),
        jax.ShapeDtypeStruct((1, HID), f32),
        jax.ShapeDtypeStruct((1, HID), f32),
        jax.ShapeDtypeStruct((1, HID), f32),
    ]
    node_embs, loc_embs, graph_emb, loc_global, latent = pl.pallas_call(
        _body, out_shape=out_shape)(*args)
    return node_embs, loc_embs, graph_emb, loc_global, latent


# speedup vs baseline: 3372.0969x; 1.0868x over previous
import jax
import jax.numpy as jnp
from jax.experimental import pallas as pl

N = 512
L = 64
HID = 128
_NEG = -1e30


def _layernorm(x, g, b):
    mu = jnp.mean(x, axis=-1, keepdims=True)
    var = jnp.mean((x - mu) ** 2, axis=-1, keepdims=True)
    return (x - mu) * jax.lax.rsqrt(var + 1e-5) * g + b


def _mm(a, b):
    return jnp.dot(a, b, preferred_element_type=jnp.float32)


def _mm_rows(xR, w):
    return jax.lax.dot_general(xR, w, (((0,), (0,)), ((), ())),
                               preferred_element_type=jnp.float32)


def _conv_layer(x, maskT, leaT, p, H, C, x_rows=False):
    proj = _mm_rows if x_rows else _mm
    inv = 1.0 / (C ** 0.5)
    q = (proj(x, p['Wq']) + p['bq']) * inv
    k = proj(x, p['Wk']) + p['bk']
    v = proj(x, p['Wv']) + p['bv']
    skip = proj(x, p['Ws']) + p['bs']
    We = p['We']
    qWe = q * We
    outs = []
    for h in range(H):
        sl = slice(h * C, (h + 1) * C)
        qh = q[:, sl]
        kh = k[:, sl]
        vh = v[:, sl]
        th = jnp.sum(qWe[:, sl], axis=1, keepdims=True)
        s = jax.lax.dot_general(qh, kh, (((1,), (1,)), ((), ())),
                                preferred_element_type=jnp.float32)
        s = jnp.where(maskT, s + leaT * th, _NEG)
        amax = jnp.max(s, axis=1, keepdims=True)
        amax = jnp.where(amax > -1e29, amax, 0.0)
        ex = jnp.exp(s - amax)
        if C < HID:
            vh_aug = jnp.concatenate(
                [vh, jnp.ones((N, 1), jnp.float32)], axis=1)
            m = _mm(ex, vh_aug)
            denom = m[:, C:C + 1]
            ohu = m[:, :C]
        else:
            denom = jnp.sum(ex, axis=1, keepdims=True)
            ohu = _mm(ex, vh)
        rec = 1.0 / (denom + 1e-16)
        oh = ohu * rec
        w = jnp.sum(ex * leaT, axis=1, keepdims=True) * rec
        outs.append(oh + w * We[:, sl])
    out = outs[0] if H == 1 else jnp.concatenate(outs, axis=1)
    Wb = p['Wb']
    wb_o = Wb[0 * HID:1 * HID, :] + Wb[2 * HID:3 * HID, :]
    wb_s = Wb[1 * HID:2 * HID, :] - Wb[2 * HID:3 * HID, :]
    beta = jax.nn.sigmoid(_mm(out, wb_o) + _mm(skip, wb_s))
    return beta * skip + (1.0 - beta) * out


_CONV_KEYS = ('Wq', 'bq', 'Wk', 'bk', 'Wv', 'bv', 'Ws', 'bs', 'We', 'Wb')


def _body(nC, nD, nid, nod, nloc, nava, lcs, lme, lnp_, adj, ea,
          *refs):
    it = iter(refs)
    convs = []
    for _ in range(3):
        convs.append({kk: next(it)[:] for kk in _CONV_KEYS})
    (gn, bn, Wl0, bl0, Wl1, bl1, gl, bl, Wf0, bf0, Wf1, bf1,
     out_node, out_loc, out_ge, out_lg, out_lat) = list(it)

    maskT = adj[:].T > 0.0
    leaT = jnp.log1p(ea[:].T)

    nfeatR = jnp.stack([nC[:], nD[:], nid[:], nod[:], nloc[:], nava[:]],
                       axis=0)
    lfeatR = jnp.stack([lcs[:], lme[:], lnp_[:]], axis=0)

    x = jax.nn.relu(_conv_layer(nfeatR, maskT, leaT, convs[0], 4, HID // 4,
                                x_rows=True))
    x = jax.nn.relu(_conv_layer(x, maskT, leaT, convs[1], 4, HID // 4))
    x = _conv_layer(x, maskT, leaT, convs[2], 1, HID)
    node_embs = _layernorm(x, gn[:], bn[:])
    out_node[:] = node_embs

    h = jax.nn.relu(_mm_rows(lfeatR, Wl0[:]) + bl0[:])
    h = _mm(h, Wl1[:]) + bl1[:]
    loc_embs = _layernorm(h, gl[:], bl[:])
    out_loc[:] = loc_embs

    graph_emb = jnp.mean(node_embs, axis=0, keepdims=True)
    loc_global = jnp.mean(loc_embs, axis=0, keepdims=True)
    out_ge[:] = graph_emb
    out_lg[:] = loc_global

    Wf0m = Wf0[:]
    z = jax.nn.relu(_mm(graph_emb, Wf0m[:HID]) + _mm(loc_global, Wf0m[HID:])
                    + bf0[:])
    out_lat[:] = _mm(z, Wf1[:]) + bf1[:]


def kernel(nodes_C, nodes_D, nodes_in_degree, nodes_out_degree, nodes_loc,
           nodes_ava, adj, edge_attr, loc_cpu_speed, loc_min_processor_EAT,
           loc_num_processor, params):
    conv_args = []
    for name in ('conv0', 'conv1', 'conv2'):
        p = params[name]
        for kk in _CONV_KEYS:
            conv_args.append(p[kk])

    args = (nodes_C, nodes_D, nodes_in_degree, nodes_out_degree, nodes_loc,
            nodes_ava, loc_cpu_speed, loc_min_processor_EAT,
            loc_num_processor, adj, edge_attr, *conv_args,
            params['ln_node']['g'], params['ln_node']['b'],
            params['loc0']['W'], params['loc0']['b'],
            params['loc1']['W'], params['loc1']['b'],
            params['ln_loc']['g'], params['ln_loc']['b'],
            params['fus0']['W'], params['fus0']['b'],
            params['fus1']['W'], params['fus1']['b'])

    f32 = jnp.float32
    out_shape = [
        jax.ShapeDtypeStruct((N, HID), f32),
        jax.ShapeDtypeStruct((L, HID), f32),
        jax.ShapeDtypeStruct((1, HID), f32),
        jax.ShapeDtypeStruct((1, HID), f32),
        jax.ShapeDtypeStruct((1, HID), f32),
    ]
    node_embs, loc_embs, graph_emb, loc_global, latent = pl.pallas_call(
        _body, out_shape=out_shape)(*args)
    return node_embs, loc_embs, graph_emb, loc_global, latent
